# baseline (speedup 1.0000x reference)
import functools

import jax
import jax.numpy as jnp
from jax import lax
from jax.experimental import pallas as pl
from jax.experimental.pallas import tpu as pltpu
from jax.experimental.pallas import tpu_sc as plsc

BATCH = 16384
DIM = 64
W = 128
LANES = 16


def _transe_kernel(h_hbm, t_hbm, r_hbm, ent_hbm, rel_hbm, o_hbm,
                   t_rows, r_rows, sem_h, sem_t, sem_r):
    def body(hi, ti, ri, o_vmem):
        ch = pltpu.async_copy(ent_hbm.at[hi.at[0]], o_vmem, sem_h)
        ct = pltpu.async_copy(ent_hbm.at[ti.at[0]], t_rows, sem_t)
        cr = pltpu.async_copy(rel_hbm.at[ri.at[0]], r_rows, sem_r)
        ch.wait()
        ct.wait()
        cr.wait()

        @pl.loop(0, W)
        def _(i):
            for j in range(DIM // LANES):
                slc = (pl.ds(i, 1), pl.ds(j * LANES, LANES))
                o_vmem.at[slc][...] = (
                    o_vmem.at[slc][...] + r_rows.at[slc][...] - t_rows.at[slc][...]
                )

    pltpu.emit_pipeline(
        body,
        grid=(BATCH // W,),
        in_specs=[
            pl.BlockSpec((1, W), index_map=lambda i: (0, i)),
            pl.BlockSpec((1, W), index_map=lambda i: (0, i)),
            pl.BlockSpec((1, W), index_map=lambda i: (0, i)),
        ],
        out_specs=[pl.BlockSpec((W, DIM), index_map=lambda i: (i, 0))],
        core_axis_name=("core", "subcore"),
        dimension_semantics=(pltpu.PARALLEL,),
    )(h_hbm, t_hbm, r_hbm, o_hbm)


@jax.jit
def kernel(h_list, t_list, r_list, ent_embeddings, rel_embeddings):
    mesh = plsc.VectorSubcoreMesh(core_axis_name="core",
                                  subcore_axis_name="subcore")
    run = pl.kernel(
        _transe_kernel,
        out_type=jax.ShapeDtypeStruct((BATCH, DIM), ent_embeddings.dtype),
        mesh=mesh,
        compiler_params=pltpu.CompilerParams(use_tc_tiling_on_sc=False),
        scratch_types=[
            pltpu.VMEM((W, DIM), ent_embeddings.dtype),
            pltpu.VMEM((W, DIM), ent_embeddings.dtype),
            pltpu.SemaphoreType.DMA,
            pltpu.SemaphoreType.DMA,
            pltpu.SemaphoreType.DMA,
        ],
    )
    return run(
        h_list.astype(jnp.int32).reshape(1, BATCH),
        t_list.astype(jnp.int32).reshape(1, BATCH),
        r_list.astype(jnp.int32).reshape(1, BATCH),
        ent_embeddings,
        rel_embeddings,
    )


# speedup vs baseline: 1.9615x; 1.9615x over previous
import jax
import jax.numpy as jnp
from jax import lax
from jax.experimental import pallas as pl
from jax.experimental.pallas import tpu as pltpu
from jax.experimental.pallas import tpu_sc as plsc

BATCH = 16384
DIM = 64
NW = 32
ROWS = BATCH // NW
C = 8
NCHUNK = ROWS // C
LANES = 16


def _transe_kernel(h_hbm, t_hbm, r_hbm, ent_hbm, rel_hbm, o_hbm,
                   ivh, ivt, ivr,
                   hbuf0, hbuf1, tbuf0, tbuf1, rbuf0, rbuf1,
                   obuf0, obuf1,
                   sem0, sem1, osem0, osem1):
    wid = lax.axis_index("core") * 16 + lax.axis_index("subcore")
    base = wid * ROWS

    pltpu.sync_copy(h_hbm.at[pl.ds(base, ROWS)], ivh.at[pl.ds(0, ROWS)])
    pltpu.sync_copy(t_hbm.at[pl.ds(base, ROWS)], ivt.at[pl.ds(0, ROWS)])
    pltpu.sync_copy(r_hbm.at[pl.ds(base, ROWS)], ivr.at[pl.ds(0, ROWS)])

    def tile_ids(rb):
        hv = ivh[pl.ds(rb, LANES)]
        tv = ivt[pl.ds(rb, LANES)]
        rv = ivr[pl.ds(rb, LANES)]
        return (lax.shift_right_logical(hv, 3),
                lax.shift_right_logical(tv, 3),
                lax.shift_right_logical(rv, 3))

    def sublanes(rb):
        return ivh[pl.ds(rb, LANES)] & 7, ivt[pl.ds(rb, LANES)] & 7, \
               ivr[pl.ds(rb, LANES)] & 7

    def issue(tids, lo, hbuf, tbuf, rbuf, sem):
        th, tt, tr = tids
        for w in range(C):
            pltpu.async_copy(ent_hbm.at[pl.ds(th[lo + w], 1)],
                             hbuf.at[pl.ds(w, 1)], sem)
            pltpu.async_copy(ent_hbm.at[pl.ds(tt[lo + w], 1)],
                             tbuf.at[pl.ds(w, 1)], sem)
            pltpu.async_copy(rel_hbm.at[pl.ds(tr[lo + w], 1)],
                             rbuf.at[pl.ds(w, 1)], sem)

    def drain(hbuf, tbuf, rbuf, sem):
        for w in range(C):
            pltpu.make_async_copy(ent_hbm.at[pl.ds(0, 1)],
                                  hbuf.at[pl.ds(w, 1)], sem).wait()
            pltpu.make_async_copy(ent_hbm.at[pl.ds(0, 1)],
                                  tbuf.at[pl.ds(w, 1)], sem).wait()
            pltpu.make_async_copy(rel_hbm.at[pl.ds(0, 1)],
                                  rbuf.at[pl.ds(w, 1)], sem).wait()

    def compute(subs, lo, g, hbuf, tbuf, rbuf, obuf, osem):
        sh, st, sr = subs
        pltpu.make_async_copy(obuf, o_hbm.at[pl.ds(base, C)], osem).wait()
        for w in range(C):
            hs = sh[lo + w]
            ts = st[lo + w]
            rs = sr[lo + w]
            for j in range(DIM // LANES):
                s = pl.ds(j * LANES, LANES)
                obuf.at[w, s][...] = (
                    hbuf.at[w, hs, s][...]
                    + rbuf.at[w, rs, s][...]
                    - tbuf.at[w, ts, s][...]
                )
        pltpu.async_copy(obuf, o_hbm.at[pl.ds(base + g * C, C)], osem)

    pltpu.async_copy(obuf0, o_hbm.at[pl.ds(base, C)], osem0)
    pltpu.async_copy(obuf1, o_hbm.at[pl.ds(base + C, C)], osem1)
    issue(tile_ids(0), 0, hbuf0, tbuf0, rbuf0, sem0)

    @pl.loop(0, NCHUNK, step=2)
    def _(g):
        rb = g * C
        tidsA = tile_ids(rb)
        subsA = sublanes(rb)
        issue(tidsA, C, hbuf1, tbuf1, rbuf1, sem1)
        drain(hbuf0, tbuf0, rbuf0, sem0)
        compute(subsA, 0, g, hbuf0, tbuf0, rbuf0, obuf0, osem0)

        @pl.when(g + 2 < NCHUNK)
        def _():
            issue(tile_ids(rb + 2 * C), 0, hbuf0, tbuf0, rbuf0, sem0)

        drain(hbuf1, tbuf1, rbuf1, sem1)
        compute(subsA, C, g + 1, hbuf1, tbuf1, rbuf1, obuf1, osem1)

    pltpu.make_async_copy(obuf0, o_hbm.at[pl.ds(base, C)], osem0).wait()
    pltpu.make_async_copy(obuf1, o_hbm.at[pl.ds(base, C)], osem1).wait()


@jax.jit
def kernel(h_list, t_list, r_list, ent_embeddings, rel_embeddings):
    n_ent, dim = ent_embeddings.shape
    n_rel = rel_embeddings.shape[0]
    mesh = plsc.VectorSubcoreMesh(core_axis_name="core",
                                  subcore_axis_name="subcore")
    fbuf = pltpu.VMEM((C, 8, DIM), ent_embeddings.dtype)
    run = pl.kernel(
        _transe_kernel,
        out_type=jax.ShapeDtypeStruct((BATCH, DIM), ent_embeddings.dtype),
        mesh=mesh,
        scratch_types=[
            pltpu.VMEM((ROWS + 2 * LANES,), jnp.int32),
            pltpu.VMEM((ROWS + 2 * LANES,), jnp.int32),
            pltpu.VMEM((ROWS + 2 * LANES,), jnp.int32),
            fbuf, fbuf, fbuf, fbuf, fbuf, fbuf,
            pltpu.VMEM((C, DIM), ent_embeddings.dtype),
            pltpu.VMEM((C, DIM), ent_embeddings.dtype),
            pltpu.SemaphoreType.DMA,
            pltpu.SemaphoreType.DMA,
            pltpu.SemaphoreType.DMA,
            pltpu.SemaphoreType.DMA,
        ],
    )
    return run(
        h_list.astype(jnp.int32),
        t_list.astype(jnp.int32),
        r_list.astype(jnp.int32),
        ent_embeddings.reshape(n_ent // 8, 8, dim),
        rel_embeddings.reshape(n_rel // 8, 8, dim),
    )
